# baseline (speedup 1.0000x reference)
import jax
import jax.numpy as jnp
from jax.experimental import pallas as pl


_BLOCK = 2048


def _body(desc_ref, z_ref, w0aug_ref, w1t_ref, out_ref):
    i = pl.program_id(0)
    q_bf = desc_ref[...].astype(jnp.bfloat16)
    z_row = z_ref[...].reshape(1, -1)
    tt = jax.lax.broadcasted_iota(jnp.int32, (128, 1), 0)
    m_lane = (tt == z_row).astype(jnp.float32)
    m_col = m_lane.T
    m_bf = m_col.astype(jnp.bfloat16)
    qexp = jnp.concatenate(
        [q_bf * m_bf[:, t:t + 1] for t in range(4)] + [m_bf], axis=1)
    a = jnp.dot(qexp, w0aug_ref[...],
                preferred_element_type=jnp.float32)
    th = jnp.tanh(a)
    r8 = jnp.dot(th, w1t_ref[...],
                 preferred_element_type=jnp.float32,
                 precision=jax.lax.Precision.HIGHEST)
    total = jnp.sum(m_col[:, :8] * r8)

    @pl.when(i == 0)
    def _():
        out_ref[...] = jnp.zeros_like(out_ref)

    out_ref[...] = out_ref[...] + total


def kernel(descriptors, gradients, grad_index, positions, Z, box, W0, b0, W1, b1):
    n, d = descriptors.shape
    t, _, h = W0.shape
    block = min(_BLOCK, n)
    nb = n // block
    z3 = Z.astype(jnp.int32).reshape(nb, 1, block)
    b0pad = jnp.zeros((d, h), jnp.float32).at[:t, :].set(b0)
    w0aug = jnp.concatenate([W0.reshape(t * d, h), b0pad], axis=0)
    w0aug_bf = w0aug.astype(jnp.bfloat16)
    w1t = jnp.zeros((h, 8), jnp.float32).at[:, :t].set(W1.T)

    out = pl.pallas_call(
        _body,
        grid=(nb,),
        in_specs=[
            pl.BlockSpec((block, d), lambda i: (i, 0)),
            pl.BlockSpec((1, 1, block), lambda i: (i, 0, 0)),
            pl.BlockSpec(((t + 1) * d, h), lambda i: (0, 0)),
            pl.BlockSpec((h, 8), lambda i: (0, 0)),
        ],
        out_specs=pl.BlockSpec((1, 1), lambda i: (0, 0)),
        out_shape=jax.ShapeDtypeStruct((1, 1), jnp.float32),
    )(descriptors, z3, w0aug_bf, w1t)
    return -(out[0, 0] + n * b1)


# speedup vs baseline: 1.7416x; 1.7416x over previous
import jax
import jax.numpy as jnp
from jax.experimental import pallas as pl


_BLOCK = 2048


def _body(desc_ref, z_ref, w0_ref, b0p_ref, w1hi_ref, w1lo_ref, out_ref):
    i = pl.program_id(0)
    q_bf = desc_ref[...].astype(jnp.bfloat16)
    z_row = z_ref[...].reshape(1, -1)
    tt = jax.lax.broadcasted_iota(jnp.int32, (8, 1), 0)
    m8 = (tt == z_row).astype(jnp.float32)
    mc = m8.T
    mc_bf = mc.astype(jnp.bfloat16)

    acc = None
    for t in range(4):
        a_t = jnp.dot(q_bf, w0_ref[t], preferred_element_type=jnp.float32)
        term = a_t * mc[:, t:t + 1]
        acc = term if acc is None else acc + term
    acc = acc + jnp.dot(mc_bf, b0p_ref[...], preferred_element_type=jnp.float32)
    th = jnp.tanh(acc)
    w1sel = (jnp.dot(mc_bf, w1hi_ref[...], preferred_element_type=jnp.float32)
             + jnp.dot(mc_bf, w1lo_ref[...], preferred_element_type=jnp.float32))
    evec = jnp.sum(th * w1sel, axis=0, keepdims=True)

    @pl.when(i == 0)
    def _():
        out_ref[...] = jnp.zeros_like(out_ref)

    out_ref[...] += evec


def kernel(descriptors, gradients, grad_index, positions, Z, box, W0, b0, W1, b1):
    n, d = descriptors.shape
    t, _, h = W0.shape
    block = min(_BLOCK, n)
    nb = n // block
    z3 = Z.astype(jnp.int32).reshape(nb, 1, block)
    w0_bf = W0.astype(jnp.bfloat16)
    b0p = jnp.zeros((8, h), jnp.float32).at[:t, :].set(b0).astype(jnp.bfloat16)
    w1hi = jnp.zeros((8, h), jnp.bfloat16).at[:t, :].set(W1.astype(jnp.bfloat16))
    w1lo = jnp.zeros((8, h), jnp.bfloat16).at[:t, :].set(
        (W1 - W1.astype(jnp.bfloat16).astype(jnp.float32)).astype(jnp.bfloat16))

    out = pl.pallas_call(
        _body,
        grid=(nb,),
        in_specs=[
            pl.BlockSpec((block, d), lambda i: (i, 0)),
            pl.BlockSpec((1, 1, block), lambda i: (i, 0, 0)),
            pl.BlockSpec((t, d, h), lambda i: (0, 0, 0)),
            pl.BlockSpec((8, h), lambda i: (0, 0)),
            pl.BlockSpec((8, h), lambda i: (0, 0)),
            pl.BlockSpec((8, h), lambda i: (0, 0)),
        ],
        out_specs=pl.BlockSpec((1, h), lambda i: (0, 0)),
        out_shape=jax.ShapeDtypeStruct((1, h), jnp.float32),
    )(descriptors, z3, w0_bf, b0p, w1hi, w1lo)
    return -(jnp.sum(out) + n * b1)


# speedup vs baseline: 2.2966x; 1.3186x over previous
import jax
import jax.numpy as jnp
from jax.experimental import pallas as pl


_BLOCK = 4096


def _body(desc_ref, z_ref, w0_ref, b0_ref, w1_ref, out_ref):
    i = pl.program_id(0)
    q_bf = desc_ref[...].astype(jnp.bfloat16)
    z_row = z_ref[...].reshape(1, -1)
    tt = jax.lax.broadcasted_iota(jnp.int32, (8, 1), 0)
    m8 = (tt == z_row).astype(jnp.float32)
    mc = m8.T
    mc_bf = mc.astype(jnp.bfloat16)

    zpad = jnp.zeros((4, 128), jnp.float32)
    b0p = jnp.concatenate([b0_ref[...], zpad], axis=0).astype(jnp.bfloat16)
    w1f = w1_ref[...]
    w1hi4 = w1f.astype(jnp.bfloat16)
    w1lo4 = (w1f - w1hi4.astype(jnp.float32)).astype(jnp.bfloat16)
    zpad_bf = zpad.astype(jnp.bfloat16)
    w1hi = jnp.concatenate([w1hi4, zpad_bf], axis=0)
    w1lo = jnp.concatenate([w1lo4, zpad_bf], axis=0)

    acc = None
    for t in range(4):
        a_t = jnp.dot(q_bf, w0_ref[t].astype(jnp.bfloat16),
                      preferred_element_type=jnp.float32)
        term = a_t * mc[:, t:t + 1]
        acc = term if acc is None else acc + term
    acc = acc + jnp.dot(mc_bf, b0p, preferred_element_type=jnp.float32)
    th = jnp.tanh(acc)
    w1sel = (jnp.dot(mc_bf, w1hi, preferred_element_type=jnp.float32)
             + jnp.dot(mc_bf, w1lo, preferred_element_type=jnp.float32))
    evec = jnp.sum(th * w1sel, axis=0, keepdims=True)

    @pl.when(i == 0)
    def _():
        out_ref[...] = jnp.zeros_like(out_ref)

    out_ref[...] += evec


def kernel(descriptors, gradients, grad_index, positions, Z, box, W0, b0, W1, b1):
    n, d = descriptors.shape
    t, _, h = W0.shape
    block = min(_BLOCK, n)
    nb = n // block
    z3 = Z.astype(jnp.int32).reshape(nb, 1, block)

    out = pl.pallas_call(
        _body,
        grid=(nb,),
        in_specs=[
            pl.BlockSpec((block, d), lambda i: (i, 0)),
            pl.BlockSpec((1, 1, block), lambda i: (i, 0, 0)),
            pl.BlockSpec((t, d, h), lambda i: (0, 0, 0)),
            pl.BlockSpec((t, h), lambda i: (0, 0)),
            pl.BlockSpec((t, h), lambda i: (0, 0)),
        ],
        out_specs=pl.BlockSpec((1, h), lambda i: (0, 0)),
        out_shape=jax.ShapeDtypeStruct((1, h), jnp.float32),
    )(descriptors, z3, W0, b0, W1)
    return -(jnp.sum(out) + n * b1)
